# baseline (speedup 1.0000x reference)
import functools

import jax
import jax.numpy as jnp
from jax import lax
from jax.experimental import pallas as pl
from jax.experimental.pallas import tpu as pltpu
from jax.experimental.pallas import tpu_sc as plsc

_DT = 7
_NW = 32
_L = 16
_MG = 4


def _dense_body(xlr_ref, xan_ref, ylr_ref, msk_ref, dy_ref, ga_ref):
    xlr = xlr_ref[...]
    dy_ref[...] = (ylr_ref[...] - xlr) * msk_ref[...]
    ga_ref[...] = xlr + xan_ref[...]


def _dense_stage(x, ylr, msk_lr):
    B, T2, H, W = x.shape
    T = T2 // 2
    grid = (B * T,)
    bs = (1, 1, H, W)
    lr_spec = pl.BlockSpec(bs, lambda i: (i // T, i % T, 0, 0))
    an_spec = pl.BlockSpec(bs, lambda i: (i // T, T + i % T, 0, 0))
    o_spec = pl.BlockSpec(bs, lambda i: (i // T, i % T, 0, 0))
    out_shape = [
        jax.ShapeDtypeStruct((B, T, H, W), jnp.float32),
        jax.ShapeDtypeStruct((B, T, H, W), jnp.float32),
    ]
    return pl.pallas_call(
        _dense_body,
        grid=grid,
        in_specs=[lr_spec, an_spec, o_spec, o_spec],
        out_specs=[o_spec, o_spec],
        out_shape=out_shape,
    )(x, x, ylr, msk_lr)


def _interp_stage(wid, table_ref, fu_h, out_h,
                  bufs, sems, pbuf, T, H, W, C, cpb, nch, tsc_row):
    cbuf, wbuf, tbuf, idxbuf, gbuf, obuf = bufs
    sem_c, sem_g, sem_o = sems
    D = len(cbuf)
    NG = C // _L
    HW = H * W
    THW = T * HW
    row_mode = table_ref.ndim == 2
    nb = table_ref.shape[0] // (HW if row_mode else THW)
    lane_i = lax.iota(jnp.int32, _L)
    gt0 = pbuf[0]
    tsc = pbuf[tsc_row]
    gy0 = pbuf[2]
    ysc = pbuf[3]
    gx0 = pbuf[4]
    xsc = pbuf[5]
    count = nch // _NW

    def base_of(k):
        return (wid + k * _NW) * C

    def fire_fu(k, s):
        pltpu.async_copy(fu_h.at[pl.ds(base_of(k) * 4, 4 * C)], cbuf[s],
                         sem_c[s])

    def do_idx(k, s):
        pltpu.make_async_copy(fu_h.at[pl.ds(0, 4 * C)], cbuf[s],
                              sem_c[s]).wait()
        cid = wid + k * _NW
        rb = jnp.minimum(cid // cpb, nb - 1) * (HW if row_mode else THW)

        def idx_pass(g, _):
            o = g * _L
            ti = (cbuf[s][pl.ds(o, _L)] - gt0) * tsc
            yi = (cbuf[s][pl.ds(C + o, _L)] - gy0) * ysc
            xi = (cbuf[s][pl.ds(2 * C + o, _L)] - gx0) * xsc
            valid = ((ti >= 0.0) & (ti <= T - 1.0)
                     & (yi >= 0.0) & (yi <= H - 1.0)
                     & (xi >= 0.0) & (xi <= W - 1.0))
            t0 = jnp.clip(ti, 0.0, T - 2.0).astype(jnp.int32)
            y0 = jnp.clip(yi, 0.0, H - 2.0).astype(jnp.int32)
            x0 = jnp.clip(xi, 0.0, W - 2.0).astype(jnp.int32)
            wt = jnp.clip(ti - t0.astype(jnp.float32), 0.0, 1.0)
            wy = jnp.clip(yi - y0.astype(jnp.float32), 0.0, 1.0)
            wx = jnp.clip(xi - x0.astype(jnp.float32), 0.0, 1.0)
            ib = idxbuf[s]
            if row_mode:
                col00 = rb + y0 * W + x0
                ib[pl.ds(0 * C + o, _L)] = col00
                ib[pl.ds(1 * C + o, _L)] = col00 + 1
                ib[pl.ds(2 * C + o, _L)] = col00 + W
                ib[pl.ds(3 * C + o, _L)] = col00 + (W + 1)
                tbuf[s][pl.ds(o, _L)] = t0
            else:
                col00 = rb + t0 * HW + y0 * W + x0
                ib[pl.ds(0 * C + o, _L)] = col00
                ib[pl.ds(1 * C + o, _L)] = col00 + 1
                ib[pl.ds(2 * C + o, _L)] = col00 + W
                ib[pl.ds(3 * C + o, _L)] = col00 + (W + 1)
                ib[pl.ds(4 * C + o, _L)] = col00 + HW
                ib[pl.ds(5 * C + o, _L)] = col00 + (HW + 1)
                ib[pl.ds(6 * C + o, _L)] = col00 + (HW + W)
                ib[pl.ds(7 * C + o, _L)] = col00 + (HW + W + 1)
            wb = wbuf[s]
            wb[pl.ds(0 * C + o, _L)] = wt
            wb[pl.ds(1 * C + o, _L)] = wy
            wb[pl.ds(2 * C + o, _L)] = wx
            wb[pl.ds(3 * C + o, _L)] = jnp.where(valid, 1.0, 0.0)
            return 0

        lax.fori_loop(0, NG, idx_pass, 0)
        pltpu.async_copy(table_ref.at[idxbuf[s]], gbuf[s], sem_g[s])

    def do_mix(k, s):
        pltpu.make_async_copy(table_ref.at[idxbuf[s]], gbuf[s],
                              sem_g[s]).wait()

        def mix_pass(g, _):
            o = g * _L
            wb = wbuf[s]
            gb = gbuf[s]
            wt = wb[pl.ds(0 * C + o, _L)]
            wy = wb[pl.ds(1 * C + o, _L)]
            wx = wb[pl.ds(2 * C + o, _L)]
            vld = wb[pl.ds(3 * C + o, _L)]
            cs = []
            if row_mode:
                t0 = tbuf[s][pl.ds(o, _L)]
                t1 = t0 + 1
                r0 = o + lane_i
                for q in range(4):
                    rq = r0 + q * C
                    v0 = plsc.load_gather(gb, [rq, t0])
                    v1 = plsc.load_gather(gb, [rq, t1])
                    cs.append(v0 * (1.0 - wt) + v1 * wt)
            else:
                for q in range(4):
                    v0 = gb[pl.ds(q * C + o, _L)]
                    v1 = gb[pl.ds((q + 4) * C + o, _L)]
                    cs.append(v0 * (1.0 - wt) + v1 * wt)
            c0 = cs[0] * (1.0 - wx) + cs[1] * wx
            c1 = cs[2] * (1.0 - wx) + cs[3] * wx
            sx_val = c0 * (1.0 - wy) + c1 * wy
            obuf[s][pl.ds(o, _L)] = (
                sx_val - cbuf[s][pl.ds(3 * C + o, _L)]) * vld
            return 0

        lax.fori_loop(0, NG, mix_pass, 0)
        pltpu.async_copy(obuf[s], out_h.at[pl.ds(base_of(k), C)], sem_o[s])

    def wait_out(s):
        pltpu.make_async_copy(obuf[s], out_h.at[pl.ds(0, C)],
                              sem_o[s]).wait()

    for j in range(min(D, count)):
        fire_fu(j, j)

    rounds = -(-(count + _MG) // D)

    def round_body(r, _):
        for s in range(D):
            i = r * D + s
            m = i - _MG
            sm = (s - _MG) % D
            pl.when((m >= D) & (m < count))(lambda sm=sm: wait_out(sm))

            def _mix(m=m, sm=sm):
                do_mix(m, sm)
            pl.when((m >= 0) & (m < count))(_mix)

            def _fu(m=m, sm=sm):
                fire_fu(m + D, sm)
            pl.when((m >= 0) & (m + D < count))(_fu)

            def _idx(i=i, s=s):
                do_idx(i, s)
            pl.when(i < count)(_idx)
        return 0

    lax.fori_loop(0, rounds, round_body, 0)

    for s in range(min(D, count)):
        wait_out(s)


_NBUF = 6


def _make_sc_kernel(B, T2, H, W, NSP, NNP, CS, CN, cpb_s, cpb_n, D1, D2, RS):
    T = T2 // 2

    mesh = plsc.VectorSubcoreMesh(core_axis_name="c", subcore_axis_name="s")

    def _stage_bufs(C, D, RW):
        if RW > 1:
            idx_t = pltpu.VMEM((4 * C,), jnp.int32)
            g_t = pltpu.VMEM((4 * C, RW), jnp.float32)
        else:
            idx_t = pltpu.VMEM((8 * C,), jnp.int32)
            g_t = pltpu.VMEM((8 * C,), jnp.float32)
        return ([pltpu.VMEM((4 * C,), jnp.float32) for _ in range(D)]
                + [pltpu.VMEM((4 * C,), jnp.float32) for _ in range(D)]
                + [pltpu.VMEM((C,), jnp.int32) for _ in range(D)]
                + [idx_t for _ in range(D)]
                + [g_t for _ in range(D)]
                + [pltpu.VMEM((C,), jnp.float32) for _ in range(D)])

    DS = max(D1, D2)

    @functools.partial(
        pl.kernel,
        out_type=[
            jax.ShapeDtypeStruct((NSP,), jnp.float32),
            jax.ShapeDtypeStruct((NNP,), jnp.float32),
        ],
        mesh=mesh,
        compiler_params=pltpu.CompilerParams(
            needs_layout_passes=False, use_tc_tiling_on_sc=False),
        scratch_types=(
            _stage_bufs(CS, D1, RS) + _stage_bufs(CN, D2, 1)
            + [pltpu.VMEM((8, _L), jnp.float32)]
            + [pltpu.SemaphoreType.DMA for _ in range(3 * DS)]
        ),
    )
    def sc_kernel(tableA_h, tableX_h, sfu_h, nfu_h, params_h,
                  dyout_h, dyout1_h, *scr):
        n1 = _NBUF * D1
        n2 = _NBUF * D2
        sbufs = [tuple(scr[i * D1:(i + 1) * D1]) for i in range(_NBUF)]
        nbufs = [tuple(scr[n1 + i * D2:n1 + (i + 1) * D2])
                 for i in range(_NBUF)]
        pbuf = scr[n1 + n2]
        allsems = scr[n1 + n2 + 1:]
        sems1 = [tuple(allsems[i * DS:i * DS + D1]) for i in range(3)]
        sems2 = [tuple(allsems[i * DS:i * DS + D2]) for i in range(3)]
        wid = lax.axis_index("s") * 2 + lax.axis_index("c")
        pltpu.sync_copy(params_h, pbuf)
        _interp_stage(wid, tableA_h, sfu_h, dyout_h,
                      sbufs, sems1, pbuf, T, H, W, CS, cpb_s,
                      NSP // CS, 1)
        _interp_stage(wid, tableX_h, nfu_h, dyout1_h,
                      nbufs, sems2, pbuf, T2, H, W, CN, cpb_n,
                      NNP // CN, 6)

    return sc_kernel


def _pad_to(a, n):
    return jnp.pad(a.reshape(-1), (0, n - a.size))


def kernel(x, ylr, msk_lr, gt, gy, gx, st, sy, sx, sv, nt, ny, nx, nv):
    B, T2, H, W = x.shape
    T = T2 // 2
    _, NT, NXs = st.shape
    NN = nt.shape[1]
    NS = B * NT * NXs
    NNF = B * NN

    dyoutlr, gridA = _dense_stage(x, ylr, msk_lr)

    tden = gt[-1] - gt[0]
    params = jnp.stack([
        gt[0], (T - 1.0) / tden, gy[0], 1.0 / (gy[1] - gy[0]),
        gx[0], 1.0 / (gx[1] - gx[0]), (T2 - 1.0) / tden, 0.0 * gt[0],
    ]).astype(jnp.float32)
    params = jnp.broadcast_to(params[:, None], (8, _L))

    CS, CN = 400, 80
    ppb_s = NT * NXs
    assert ppb_s % CS == 0
    assert NN % CN == 0
    nch_s = -(-NS // CS)
    nch_s += (-nch_s) % _NW
    NSP = nch_s * CS
    nch_n = -(-NNF // CN)
    nch_n += (-nch_n) % _NW
    NNP = nch_n * CN
    D1 = min(6, nch_s // _NW)
    D2 = min(8, nch_n // _NW)
    RS = 8
    assert T <= RS

    def _fuse(arrs, NP, C):
        cols = jnp.stack([_pad_to(a, NP).reshape(NP // C, C) for a in arrs],
                         axis=1)
        return cols.reshape(-1)

    sfu = _fuse((st, sy, sx, sv), NSP, CS)
    nfu = _fuse((nt, ny, nx, nv), NNP, CN)

    tabA = jnp.concatenate(
        [jnp.moveaxis(gridA, 1, -1),
         jnp.zeros((B, H, W, RS - T), jnp.float32)], -1).reshape(-1, RS)

    sc = _make_sc_kernel(B, T2, H, W, NSP, NNP, CS, CN,
                         ppb_s // CS, NN // CN, D1, D2, RS)
    dyout_flat, dyout1_flat = sc(tabA, x.reshape(-1), sfu, nfu, params)

    return (dyoutlr,
            dyout_flat[:NS].reshape(B, NT, NXs),
            dyout1_flat[:NNF].reshape(B, NN))


# speedup vs baseline: 1.3938x; 1.3938x over previous
import functools

import jax
import jax.numpy as jnp
from jax import lax
from jax.experimental import pallas as pl
from jax.experimental.pallas import tpu as pltpu
from jax.experimental.pallas import tpu_sc as plsc

_DT = 7
_NW = 32
_L = 16
_MG = 4


def _dense_body(xlr_ref, xan_ref, ylr_ref, msk_ref, dy_ref, ga_ref):
    xlr = xlr_ref[...]
    dy_ref[...] = (ylr_ref[...] - xlr) * msk_ref[...]
    ga_ref[...] = xlr + xan_ref[...]


def _dense_stage(x, ylr, msk_lr):
    B, T2, H, W = x.shape
    T = T2 // 2
    grid = (B * T,)
    bs = (1, 1, H, W)
    lr_spec = pl.BlockSpec(bs, lambda i: (i // T, i % T, 0, 0))
    an_spec = pl.BlockSpec(bs, lambda i: (i // T, T + i % T, 0, 0))
    o_spec = pl.BlockSpec(bs, lambda i: (i // T, i % T, 0, 0))
    out_shape = [
        jax.ShapeDtypeStruct((B, T, H, W), jnp.float32),
        jax.ShapeDtypeStruct((B, T, H, W), jnp.float32),
    ]
    return pl.pallas_call(
        _dense_body,
        grid=grid,
        in_specs=[lr_spec, an_spec, o_spec, o_spec],
        out_specs=[o_spec, o_spec],
        out_shape=out_shape,
    )(x, x, ylr, msk_lr)


def _interp_stage(wid, table_ref, fu_h, out_h,
                  bufs, sems, pbuf, T, H, W, C, cpb, nch, tsc_row):
    cbuf, wbuf, tbuf, idxbuf, gbuf, obuf = bufs
    sem_c, sem_g, sem_o = sems
    D = len(cbuf)
    NG = C // _L
    HW = H * W
    THW = T * HW
    row_mode = table_ref.ndim == 2
    nb = table_ref.shape[0] // (HW if row_mode else THW)
    lane_i = lax.iota(jnp.int32, _L)
    gt0 = pbuf[0]
    tsc = pbuf[tsc_row]
    gy0 = pbuf[2]
    ysc = pbuf[3]
    gx0 = pbuf[4]
    xsc = pbuf[5]
    count = nch // _NW

    def base_of(k):
        return (wid + k * _NW) * C

    def fire_fu(k, s):
        pltpu.async_copy(fu_h.at[pl.ds(base_of(k) * 4, 4 * C)], cbuf[s],
                         sem_c[s])

    def do_idx(k, s):
        pltpu.make_async_copy(fu_h.at[pl.ds(0, 4 * C)], cbuf[s],
                              sem_c[s]).wait()
        cid = wid + k * _NW
        rb = jnp.minimum(cid // cpb, nb - 1) * (HW if row_mode else THW)

        def idx_pass(g, _):
            o = g * _L
            ti = (cbuf[s][pl.ds(o, _L)] - gt0) * tsc
            yi = (cbuf[s][pl.ds(C + o, _L)] - gy0) * ysc
            xi = (cbuf[s][pl.ds(2 * C + o, _L)] - gx0) * xsc
            valid = ((ti >= 0.0) & (ti <= T - 1.0)
                     & (yi >= 0.0) & (yi <= H - 1.0)
                     & (xi >= 0.0) & (xi <= W - 1.0))
            t0 = jnp.clip(ti, 0.0, T - 2.0).astype(jnp.int32)
            y0 = jnp.clip(yi, 0.0, H - 2.0).astype(jnp.int32)
            x0 = jnp.clip(xi, 0.0, W - 2.0).astype(jnp.int32)
            wt = jnp.clip(ti - t0.astype(jnp.float32), 0.0, 1.0)
            wy = jnp.clip(yi - y0.astype(jnp.float32), 0.0, 1.0)
            wx = jnp.clip(xi - x0.astype(jnp.float32), 0.0, 1.0)
            ib = idxbuf[s]
            if row_mode:
                ib[pl.ds(o, _L)] = rb + y0 * W + x0
                tbuf[s][pl.ds(o, _L)] = t0
            else:
                col00 = rb + t0 * HW + y0 * W + x0
                ib[pl.ds(0 * C + o, _L)] = col00
                ib[pl.ds(1 * C + o, _L)] = col00 + 1
                ib[pl.ds(2 * C + o, _L)] = col00 + W
                ib[pl.ds(3 * C + o, _L)] = col00 + (W + 1)
                ib[pl.ds(4 * C + o, _L)] = col00 + HW
                ib[pl.ds(5 * C + o, _L)] = col00 + (HW + 1)
                ib[pl.ds(6 * C + o, _L)] = col00 + (HW + W)
                ib[pl.ds(7 * C + o, _L)] = col00 + (HW + W + 1)
            wb = wbuf[s]
            wb[pl.ds(0 * C + o, _L)] = wt
            wb[pl.ds(1 * C + o, _L)] = wy
            wb[pl.ds(2 * C + o, _L)] = wx
            wb[pl.ds(3 * C + o, _L)] = jnp.where(valid, 1.0, 0.0)
            return 0

        lax.fori_loop(0, NG, idx_pass, 0)
        pltpu.async_copy(table_ref.at[idxbuf[s]], gbuf[s], sem_g[s])

    def do_mix(k, s):
        pltpu.make_async_copy(table_ref.at[idxbuf[s]], gbuf[s],
                              sem_g[s]).wait()

        def mix_pass(g, _):
            o = g * _L
            wb = wbuf[s]
            gb = gbuf[s]
            wt = wb[pl.ds(0 * C + o, _L)]
            wy = wb[pl.ds(1 * C + o, _L)]
            wx = wb[pl.ds(2 * C + o, _L)]
            vld = wb[pl.ds(3 * C + o, _L)]
            cs = []
            if row_mode:
                t0 = tbuf[s][pl.ds(o, _L)]
                rq = o + lane_i
                for q in range(4):
                    c_q = q * T + t0
                    v0 = plsc.load_gather(gb, [rq, c_q])
                    v1 = plsc.load_gather(gb, [rq, c_q + 1])
                    cs.append(v0 * (1.0 - wt) + v1 * wt)
            else:
                for q in range(4):
                    v0 = gb[pl.ds(q * C + o, _L)]
                    v1 = gb[pl.ds((q + 4) * C + o, _L)]
                    cs.append(v0 * (1.0 - wt) + v1 * wt)
            c0 = cs[0] * (1.0 - wx) + cs[1] * wx
            c1 = cs[2] * (1.0 - wx) + cs[3] * wx
            sx_val = c0 * (1.0 - wy) + c1 * wy
            obuf[s][pl.ds(o, _L)] = (
                sx_val - cbuf[s][pl.ds(3 * C + o, _L)]) * vld
            return 0

        lax.fori_loop(0, NG, mix_pass, 0)
        pltpu.async_copy(obuf[s], out_h.at[pl.ds(base_of(k), C)], sem_o[s])

    def wait_out(s):
        pltpu.make_async_copy(obuf[s], out_h.at[pl.ds(0, C)],
                              sem_o[s]).wait()

    for j in range(min(D, count)):
        fire_fu(j, j)

    rounds = -(-(count + _MG) // D)

    def round_body(r, _):
        for s in range(D):
            i = r * D + s
            m = i - _MG
            sm = (s - _MG) % D
            pl.when((m >= D) & (m < count))(lambda sm=sm: wait_out(sm))

            def _mix(m=m, sm=sm):
                do_mix(m, sm)
            pl.when((m >= 0) & (m < count))(_mix)

            def _fu(m=m, sm=sm):
                fire_fu(m + D, sm)
            pl.when((m >= 0) & (m + D < count))(_fu)

            def _idx(i=i, s=s):
                do_idx(i, s)
            pl.when(i < count)(_idx)
        return 0

    lax.fori_loop(0, rounds, round_body, 0)

    for s in range(min(D, count)):
        wait_out(s)


_NBUF = 6


def _make_sc_kernel(B, T2, H, W, NSP, NNP, CS, CN, cpb_s, cpb_n, D1, D2, RS):
    T = T2 // 2

    mesh = plsc.VectorSubcoreMesh(core_axis_name="c", subcore_axis_name="s")

    def _stage_bufs(C, D, RW):
        if RW > 1:
            idx_t = pltpu.VMEM((C,), jnp.int32)
            g_t = pltpu.VMEM((C, RW), jnp.float32)
        else:
            idx_t = pltpu.VMEM((8 * C,), jnp.int32)
            g_t = pltpu.VMEM((8 * C,), jnp.float32)
        return ([pltpu.VMEM((4 * C,), jnp.float32) for _ in range(D)]
                + [pltpu.VMEM((4 * C,), jnp.float32) for _ in range(D)]
                + [pltpu.VMEM((C,), jnp.int32) for _ in range(D)]
                + [idx_t for _ in range(D)]
                + [g_t for _ in range(D)]
                + [pltpu.VMEM((C,), jnp.float32) for _ in range(D)])

    DS = max(D1, D2)

    @functools.partial(
        pl.kernel,
        out_type=[
            jax.ShapeDtypeStruct((NSP,), jnp.float32),
            jax.ShapeDtypeStruct((NNP,), jnp.float32),
        ],
        mesh=mesh,
        compiler_params=pltpu.CompilerParams(
            needs_layout_passes=False, use_tc_tiling_on_sc=False),
        scratch_types=(
            _stage_bufs(CS, D1, RS) + _stage_bufs(CN, D2, 1)
            + [pltpu.VMEM((8, _L), jnp.float32)]
            + [pltpu.SemaphoreType.DMA for _ in range(3 * DS)]
        ),
    )
    def sc_kernel(tableA_h, tableX_h, sfu_h, nfu_h, params_h,
                  dyout_h, dyout1_h, *scr):
        n1 = _NBUF * D1
        n2 = _NBUF * D2
        sbufs = [tuple(scr[i * D1:(i + 1) * D1]) for i in range(_NBUF)]
        nbufs = [tuple(scr[n1 + i * D2:n1 + (i + 1) * D2])
                 for i in range(_NBUF)]
        pbuf = scr[n1 + n2]
        allsems = scr[n1 + n2 + 1:]
        sems1 = [tuple(allsems[i * DS:i * DS + D1]) for i in range(3)]
        sems2 = [tuple(allsems[i * DS:i * DS + D2]) for i in range(3)]
        wid = lax.axis_index("s") * 2 + lax.axis_index("c")
        pltpu.sync_copy(params_h, pbuf)
        _interp_stage(wid, tableA_h, sfu_h, dyout_h,
                      sbufs, sems1, pbuf, T, H, W, CS, cpb_s,
                      NSP // CS, 1)
        _interp_stage(wid, tableX_h, nfu_h, dyout1_h,
                      nbufs, sems2, pbuf, T2, H, W, CN, cpb_n,
                      NNP // CN, 6)

    return sc_kernel


def _pad_to(a, n):
    return jnp.pad(a.reshape(-1), (0, n - a.size))


def kernel(x, ylr, msk_lr, gt, gy, gx, st, sy, sx, sv, nt, ny, nx, nv):
    B, T2, H, W = x.shape
    T = T2 // 2
    _, NT, NXs = st.shape
    NN = nt.shape[1]
    NS = B * NT * NXs
    NNF = B * NN

    dyoutlr, gridA = _dense_stage(x, ylr, msk_lr)

    tden = gt[-1] - gt[0]
    params = jnp.stack([
        gt[0], (T - 1.0) / tden, gy[0], 1.0 / (gy[1] - gy[0]),
        gx[0], 1.0 / (gx[1] - gx[0]), (T2 - 1.0) / tden, 0.0 * gt[0],
    ]).astype(jnp.float32)
    params = jnp.broadcast_to(params[:, None], (8, _L))

    CS, CN = 320, 80
    ppb_s = NT * NXs
    assert ppb_s % CS == 0
    assert NN % CN == 0
    nch_s = -(-NS // CS)
    nch_s += (-nch_s) % _NW
    NSP = nch_s * CS
    nch_n = -(-NNF // CN)
    nch_n += (-nch_n) % _NW
    NNP = nch_n * CN
    D1 = min(7, nch_s // _NW)
    D2 = min(8, nch_n // _NW)
    RS = 32
    assert 4 * T <= RS

    def _fuse(arrs, NP, C):
        cols = jnp.stack([_pad_to(a, NP).reshape(NP // C, C) for a in arrs],
                         axis=1)
        return cols.reshape(-1)

    sfu = _fuse((st, sy, sx, sv), NSP, CS)
    nfu = _fuse((nt, ny, nx, nv), NNP, CN)

    base = jnp.moveaxis(gridA, 1, -1)
    shx = jnp.concatenate([base[:, :, 1:], base[:, :, -1:]], axis=2)
    shy = jnp.concatenate([base[:, 1:], base[:, -1:]], axis=1)
    shxy = jnp.concatenate([shy[:, :, 1:], shy[:, :, -1:]], axis=2)
    tabA = jnp.concatenate(
        [base, shx, shy, shxy,
         jnp.zeros((B, H, W, RS - 4 * T), jnp.float32)], -1).reshape(-1, RS)

    sc = _make_sc_kernel(B, T2, H, W, NSP, NNP, CS, CN,
                         ppb_s // CS, NN // CN, D1, D2, RS)
    dyout_flat, dyout1_flat = sc(tabA, x.reshape(-1), sfu, nfu, params)

    return (dyoutlr,
            dyout_flat[:NS].reshape(B, NT, NXs),
            dyout1_flat[:NNF].reshape(B, NN))
